# baseline (speedup 1.0000x reference)
import functools

import jax
import jax.numpy as jnp
from jax.experimental import pallas as pl
from jax.experimental.pallas import tpu as pltpu

_STEPS = 3
_NSPLIT = 2


def _body(*refs):
    rep_refs = refs[:_NSPLIT]
    maskf_ref, wih_ref, whh_ref, b_ref, wout_ref, bout_ref, y_ref = \
        refs[_NSPLIT:]
    reps = [r[...] for r in rep_refs]
    rep_ts = [jnp.swapaxes(r, 1, 2) for r in reps]
    maskf = maskf_ref[...]
    bb, ns, d = reps[0].shape
    q_star = jnp.zeros((bb, 2 * d), jnp.float32)
    h = jnp.zeros((bb, d), jnp.float32)
    c = jnp.zeros((bb, d), jnp.float32)
    bias = b_ref[...]
    for _ in range(_STEPS):
        gates = (jnp.dot(q_star, wih_ref[...],
                         preferred_element_type=jnp.float32)
                 + jnp.dot(h, whh_ref[...],
                           preferred_element_type=jnp.float32)
                 + bias)
        gi = jax.nn.sigmoid(gates[:, 0 * d:1 * d])
        gf = jax.nn.sigmoid(gates[:, 1 * d:2 * d])
        gg = jnp.tanh(gates[:, 2 * d:3 * d])
        go = jax.nn.sigmoid(gates[:, 3 * d:4 * d])
        c = gf * c + gi * gg
        h = go * jnp.tanh(c)
        e = jnp.concatenate(
            [jax.lax.dot_general(h, rt, (((1,), (1,)), ((0,), (0,))),
                                 preferred_element_type=jnp.float32)
             for rt in rep_ts], axis=1)
        e = jnp.where(maskf > 0, e, -jnp.inf)
        e = e - jnp.max(e, axis=1, keepdims=True)
        a = jnp.exp(e) * maskf
        a = a / jnp.sum(a, axis=1, keepdims=True)
        r = sum(
            jax.lax.dot_general(a[:, k * ns:(k + 1) * ns], reps[k],
                                (((1,), (1,)), ((0,), (0,))),
                                preferred_element_type=jnp.float32)
            for k in range(_NSPLIT))
        q_star = jnp.concatenate([h, r], axis=-1)
    y = jnp.dot(q_star, wout_ref[...],
                preferred_element_type=jnp.float32) + bout_ref[...]
    y_ref[...] = y


@functools.partial(jax.jit, static_argnames=("interpret",))
def kernel(representation, atom_mask, W_ih, W_hh, b_ih, b_hh, W_out, b_out,
           mean, stddev, interpret=False):
    b, n, d = representation.shape
    bb = 32
    ns = n // _NSPLIT
    maskf = atom_mask.astype(jnp.float32)
    wih_t = W_ih.T
    whh_t = W_hh.T
    bias = (b_ih + b_hh)[None, :]
    wout_t = W_out.T
    bout = b_out[None, :]

    def rep_spec(k):
        return pl.BlockSpec((bb, ns, d), lambda i, k=k: (i, k, 0))

    y = pl.pallas_call(
        _body,
        grid=(b // bb,),
        in_specs=[rep_spec(k) for k in range(_NSPLIT)] + [
            pl.BlockSpec((bb, n), lambda i: (i, 0)),
            pl.BlockSpec(wih_t.shape, lambda i: (0, 0)),
            pl.BlockSpec(whh_t.shape, lambda i: (0, 0)),
            pl.BlockSpec(bias.shape, lambda i: (0, 0)),
            pl.BlockSpec(wout_t.shape, lambda i: (0, 0)),
            pl.BlockSpec(bout.shape, lambda i: (0, 0)),
        ],
        out_specs=pl.BlockSpec((bb, 1), lambda i: (i, 0)),
        out_shape=jax.ShapeDtypeStruct((b, 1), jnp.float32),
        interpret=interpret,
    )(*([representation] * _NSPLIT),
      maskf, wih_t, whh_t, bias, wout_t, bout)
    return y * stddev + mean


# speedup vs baseline: 1.0144x; 1.0144x over previous
import functools

import jax
import jax.numpy as jnp
from jax.experimental import pallas as pl
from jax.experimental.pallas import tpu as pltpu

_STEPS = 3
_NSPLIT = 2


def _body(*refs):
    rep_refs = refs[:_NSPLIT]
    maskf_ref, wih_ref, whh_ref, b_ref, wout_ref, bout_ref, y_ref = \
        refs[_NSPLIT:]
    reps = [r[...] for r in rep_refs]
    rep_ts = [jnp.swapaxes(r, 1, 2) for r in reps]
    maskf = maskf_ref[...]
    bb, ns, d = reps[0].shape
    q_star = jnp.zeros((bb, 2 * d), jnp.float32)
    h = jnp.zeros((bb, d), jnp.float32)
    c = jnp.zeros((bb, d), jnp.float32)
    bias = b_ref[...]
    for _ in range(_STEPS):
        gates = (jnp.dot(q_star, wih_ref[...],
                         preferred_element_type=jnp.float32)
                 + jnp.dot(h, whh_ref[...],
                           preferred_element_type=jnp.float32)
                 + bias)
        gi = jax.nn.sigmoid(gates[:, 0 * d:1 * d])
        gf = jax.nn.sigmoid(gates[:, 1 * d:2 * d])
        gg = jnp.tanh(gates[:, 2 * d:3 * d])
        go = jax.nn.sigmoid(gates[:, 3 * d:4 * d])
        c = gf * c + gi * gg
        h = go * jnp.tanh(c)
        e = jnp.concatenate(
            [jax.lax.dot_general(h, rt, (((1,), (1,)), ((0,), (0,))),
                                 preferred_element_type=jnp.float32)
             for rt in rep_ts], axis=1)
        e = jnp.where(maskf > 0, e, -jnp.inf)
        e = e - jnp.max(e, axis=1, keepdims=True)
        a = jnp.exp(e) * maskf
        a = a / jnp.sum(a, axis=1, keepdims=True)
        r = sum(
            jax.lax.dot_general(a[:, k * ns:(k + 1) * ns], reps[k],
                                (((1,), (1,)), ((0,), (0,))),
                                precision=jax.lax.Precision.DEFAULT,
                                preferred_element_type=jnp.float32)
            for k in range(_NSPLIT))
        q_star = jnp.concatenate([h, r], axis=-1)
    y = jnp.dot(q_star, wout_ref[...],
                preferred_element_type=jnp.float32) + bout_ref[...]
    y_ref[...] = y


@functools.partial(jax.jit, static_argnames=("interpret",))
def kernel(representation, atom_mask, W_ih, W_hh, b_ih, b_hh, W_out, b_out,
           mean, stddev, interpret=False):
    b, n, d = representation.shape
    bb = 32
    ns = n // _NSPLIT
    maskf = atom_mask.astype(jnp.float32)
    wih_t = W_ih.T
    whh_t = W_hh.T
    bias = (b_ih + b_hh)[None, :]
    wout_t = W_out.T
    bout = b_out[None, :]

    def rep_spec(k):
        return pl.BlockSpec((bb, ns, d), lambda i, k=k: (i, k, 0))

    y = pl.pallas_call(
        _body,
        grid=(b // bb,),
        in_specs=[rep_spec(k) for k in range(_NSPLIT)] + [
            pl.BlockSpec((bb, n), lambda i: (i, 0)),
            pl.BlockSpec(wih_t.shape, lambda i: (0, 0)),
            pl.BlockSpec(whh_t.shape, lambda i: (0, 0)),
            pl.BlockSpec(bias.shape, lambda i: (0, 0)),
            pl.BlockSpec(wout_t.shape, lambda i: (0, 0)),
            pl.BlockSpec(bout.shape, lambda i: (0, 0)),
        ],
        out_specs=pl.BlockSpec((bb, 1), lambda i: (i, 0)),
        out_shape=jax.ShapeDtypeStruct((b, 1), jnp.float32),
        interpret=interpret,
    )(*([representation] * _NSPLIT),
      maskf, wih_t, whh_t, bias, wout_t, bout)
    return y * stddev + mean
